# baseline (speedup 1.0000x reference)
import functools

import jax
import jax.numpy as jnp
from jax import lax
from jax.experimental import pallas as pl
from jax.experimental.pallas import tpu as pltpu
from jax.experimental.pallas import tpu_sc as plsc


def _sc_gather(edge_idx, drug_emb, disease_emb, chunk_base=0, chunk_e=None):
    e = chunk_e if chunk_e is not None else edge_idx.shape[1]
    d = drug_emb.shape[1]
    info = plsc.get_sparse_core_info()
    nc, ns = info.num_cores, info.num_subcores
    nw = nc * ns
    b = e // nw
    hb = b // 2
    mesh = plsc.VectorSubcoreMesh(core_axis_name="c", subcore_axis_name="s")

    @functools.partial(
        pl.kernel,
        mesh=mesh,
        out_type=(
            jax.ShapeDtypeStruct((e, d), jnp.float32),
            jax.ShapeDtypeStruct((e, d), jnp.float32),
        ),
        scratch_types=[
            pltpu.VMEM((b,), jnp.int32),
            pltpu.VMEM((b,), jnp.int32),
            pltpu.VMEM((hb, d), jnp.float32),
            pltpu.VMEM((hb, d), jnp.float32),
            pltpu.SemaphoreType.DMA,
            pltpu.SemaphoreType.DMA,
            pltpu.SemaphoreType.DMA,
            pltpu.SemaphoreType.DMA,
        ],
    )
    def gather_kernel(drug_hbm, dis_hbm, idx_hbm, d_out, s_out,
                      idx_d, idx_s, buf_a, buf_b, sem_a, sem_b, sem_wa, sem_wb):
        wid = lax.axis_index("s") * nc + lax.axis_index("c")
        base = wid * b
        pltpu.sync_copy(idx_hbm.at[0, pl.ds(chunk_base + base, b)], idx_d)
        pltpu.sync_copy(idx_hbm.at[1, pl.ds(chunk_base + base, b)], idx_s)
        ga0 = pltpu.async_copy(drug_hbm.at[idx_d.at[pl.ds(0, hb)]], buf_a, sem_a)
        gb0 = pltpu.async_copy(drug_hbm.at[idx_d.at[pl.ds(hb, hb)]], buf_b, sem_b)
        ga0.wait()
        wa0 = pltpu.async_copy(buf_a, d_out.at[pl.ds(base, hb)], sem_wa)
        gb0.wait()
        wb0 = pltpu.async_copy(buf_b, d_out.at[pl.ds(base + hb, hb)], sem_wb)
        wa0.wait()
        ga1 = pltpu.async_copy(dis_hbm.at[idx_s.at[pl.ds(0, hb)]], buf_a, sem_a)
        wb0.wait()
        gb1 = pltpu.async_copy(dis_hbm.at[idx_s.at[pl.ds(hb, hb)]], buf_b, sem_b)
        ga1.wait()
        wa1 = pltpu.async_copy(buf_a, s_out.at[pl.ds(base, hb)], sem_wa)
        gb1.wait()
        wb1 = pltpu.async_copy(buf_b, s_out.at[pl.ds(base + hb, hb)], sem_wb)
        wa1.wait()
        wb1.wait()

    return gather_kernel(drug_emb, disease_emb, edge_idx)


def _sigmoid(x):
    return 0.5 * jnp.tanh(0.5 * x) + 0.5


def _mlp_block_kernel(dlo_ref, dhi_ref, slo_ref, shi_ref,
                      w1_ref, b1_ref, w2_ref, b2_ref, o_ref):
    dim = dlo_ref.shape[1]
    w1 = w1_ref[...].astype(jnp.bfloat16)
    w1a, w1b = w1[:dim], w1[dim:]
    b1v = b1_ref[...][None, :]
    w2v = w2_ref[...]
    b2v = b2_ref[0]

    def head(d_ref, s_ref):
        d_bf = d_ref[...].astype(jnp.bfloat16)
        s_bf = s_ref[...].astype(jnp.bfloat16)
        x = jnp.dot(d_bf, w1a, preferred_element_type=jnp.float32)
        x = x + jnp.dot(s_bf, w1b, preferred_element_type=jnp.float32)
        h = _sigmoid(x + b1v)
        y = jnp.dot(h, w2v, preferred_element_type=jnp.float32) + b2v
        return _sigmoid(y.T)

    lo = head(dlo_ref, slo_ref)
    hi = head(dhi_ref, shi_ref)
    o_ref[...] = jnp.concatenate([lo, hi], axis=0)


def _mlp(d_rows, s_rows, W1, b1, W2, b2, block_e=2048):
    e, dim = d_rows.shape
    half = e // 2
    nlo = half // block_e
    grid = (nlo,)
    out = pl.pallas_call(
        _mlp_block_kernel,
        grid=grid,
        in_specs=[
            pl.BlockSpec((block_e, dim), lambda i: (i, 0)),
            pl.BlockSpec((block_e, dim), lambda i: (i + nlo, 0)),
            pl.BlockSpec((block_e, dim), lambda i: (i, 0)),
            pl.BlockSpec((block_e, dim), lambda i: (i + nlo, 0)),
            pl.BlockSpec(W1.shape, lambda i: (0, 0)),
            pl.BlockSpec(b1.shape, lambda i: (0,)),
            pl.BlockSpec(W2.shape, lambda i: (0, 0)),
            pl.BlockSpec(b2.shape, lambda i: (0,)),
        ],
        out_specs=pl.BlockSpec((2, block_e), lambda i: (0, i)),
        out_shape=jax.ShapeDtypeStruct((2, half), jnp.float32),
        compiler_params=pltpu.CompilerParams(
            dimension_semantics=("parallel",),
        ),
    )(d_rows, d_rows, s_rows, s_rows, W1, b1, W2, b2)
    return out


def kernel(edge_index, drug_emb, disease_emb, W1, b1, W2, b2):
    edge_idx = edge_index.astype(jnp.int32)
    e = edge_index.shape[1]
    n_chunks = 2
    chunk_e = e // n_chunks
    outs = []
    for c in range(n_chunks):
        d_rows, s_rows = _sc_gather(edge_idx, drug_emb, disease_emb,
                                    c * chunk_e, chunk_e)
        outs.append(_mlp(d_rows, s_rows, W1, b1, W2, b2).reshape(-1))
    return jnp.concatenate(outs).reshape(-1, 1)


# speedup vs baseline: 1.0793x; 1.0793x over previous
import functools

import jax
import jax.numpy as jnp
from jax import lax
from jax.experimental import pallas as pl
from jax.experimental.pallas import tpu as pltpu
from jax.experimental.pallas import tpu_sc as plsc


def _sc_gather(edge_idx, drug_emb, disease_emb, chunk_base=0, chunk_e=None):
    e = chunk_e if chunk_e is not None else edge_idx.shape[1]
    d = drug_emb.shape[1]
    info = plsc.get_sparse_core_info()
    nc, ns = info.num_cores, info.num_subcores
    nw = nc * ns
    b = e // nw
    hb = b // 2
    mesh = plsc.VectorSubcoreMesh(core_axis_name="c", subcore_axis_name="s")

    @functools.partial(
        pl.kernel,
        mesh=mesh,
        out_type=(
            jax.ShapeDtypeStruct((e, d), jnp.float32),
            jax.ShapeDtypeStruct((e, d), jnp.float32),
        ),
        scratch_types=[
            pltpu.VMEM((b,), jnp.int32),
            pltpu.VMEM((b,), jnp.int32),
            pltpu.VMEM((hb, d), jnp.float32),
            pltpu.VMEM((hb, d), jnp.float32),
            pltpu.SemaphoreType.DMA,
            pltpu.SemaphoreType.DMA,
            pltpu.SemaphoreType.DMA,
            pltpu.SemaphoreType.DMA,
        ],
    )
    def gather_kernel(drug_hbm, dis_hbm, idx_hbm, d_out, s_out,
                      idx_d, idx_s, buf_a, buf_b, sem_a, sem_b, sem_wa, sem_wb):
        wid = lax.axis_index("s") * nc + lax.axis_index("c")
        base = wid * b
        pltpu.sync_copy(idx_hbm.at[0, pl.ds(chunk_base + base, b)], idx_d)
        pltpu.sync_copy(idx_hbm.at[1, pl.ds(chunk_base + base, b)], idx_s)
        ga0 = pltpu.async_copy(drug_hbm.at[idx_d.at[pl.ds(0, hb)]], buf_a, sem_a)
        gb0 = pltpu.async_copy(drug_hbm.at[idx_d.at[pl.ds(hb, hb)]], buf_b, sem_b)
        ga0.wait()
        wa0 = pltpu.async_copy(buf_a, d_out.at[pl.ds(base, hb)], sem_wa)
        gb0.wait()
        wb0 = pltpu.async_copy(buf_b, d_out.at[pl.ds(base + hb, hb)], sem_wb)
        wa0.wait()
        ga1 = pltpu.async_copy(dis_hbm.at[idx_s.at[pl.ds(0, hb)]], buf_a, sem_a)
        wb0.wait()
        gb1 = pltpu.async_copy(dis_hbm.at[idx_s.at[pl.ds(hb, hb)]], buf_b, sem_b)
        ga1.wait()
        wa1 = pltpu.async_copy(buf_a, s_out.at[pl.ds(base, hb)], sem_wa)
        gb1.wait()
        wb1 = pltpu.async_copy(buf_b, s_out.at[pl.ds(base + hb, hb)], sem_wb)
        wa1.wait()
        wb1.wait()

    return gather_kernel(drug_emb, disease_emb, edge_idx)


def _sigmoid(x):
    return 0.5 * jnp.tanh(0.5 * x) + 0.5


def _mlp_block_kernel(dlo_ref, dhi_ref, slo_ref, shi_ref,
                      w1_ref, b1_ref, w2_ref, b2_ref, o_ref):
    dim = dlo_ref.shape[1]
    w1 = w1_ref[...].astype(jnp.bfloat16)
    w1a, w1b = w1[:dim], w1[dim:]
    b1v = b1_ref[...][None, :]
    w2v = w2_ref[...]
    b2v = b2_ref[0]

    def head(d_ref, s_ref):
        d_bf = d_ref[...].astype(jnp.bfloat16)
        s_bf = s_ref[...].astype(jnp.bfloat16)
        x = jnp.dot(d_bf, w1a, preferred_element_type=jnp.float32)
        x = x + jnp.dot(s_bf, w1b, preferred_element_type=jnp.float32)
        h = _sigmoid(x + b1v)
        y = jnp.dot(h, w2v, preferred_element_type=jnp.float32) + b2v
        return _sigmoid(y.T)

    lo = head(dlo_ref, slo_ref)
    hi = head(dhi_ref, shi_ref)
    o_ref[...] = jnp.concatenate([lo, hi], axis=0)


def _mlp(d_rows, s_rows, W1, b1, W2, b2, block_e=2048):
    e, dim = d_rows.shape
    half = e // 2
    nlo = half // block_e
    grid = (nlo,)
    out = pl.pallas_call(
        _mlp_block_kernel,
        grid=grid,
        in_specs=[
            pl.BlockSpec((block_e, dim), lambda i: (i, 0)),
            pl.BlockSpec((block_e, dim), lambda i: (i + nlo, 0)),
            pl.BlockSpec((block_e, dim), lambda i: (i, 0)),
            pl.BlockSpec((block_e, dim), lambda i: (i + nlo, 0)),
            pl.BlockSpec(W1.shape, lambda i: (0, 0)),
            pl.BlockSpec(b1.shape, lambda i: (0,)),
            pl.BlockSpec(W2.shape, lambda i: (0, 0)),
            pl.BlockSpec(b2.shape, lambda i: (0,)),
        ],
        out_specs=pl.BlockSpec((2, block_e), lambda i: (0, i)),
        out_shape=jax.ShapeDtypeStruct((2, half), jnp.float32),
        compiler_params=pltpu.CompilerParams(
            dimension_semantics=("parallel",),
        ),
    )(d_rows, d_rows, s_rows, s_rows, W1, b1, W2, b2)
    return out


def kernel(edge_index, drug_emb, disease_emb, W1, b1, W2, b2):
    edge_idx = edge_index.astype(jnp.int32)
    d_rows, s_rows = _sc_gather(edge_idx, drug_emb, disease_emb)
    return _mlp(d_rows, s_rows, W1, b1, W2, b2).reshape(-1, 1)
